# baseline (speedup 1.0000x reference)
import functools

import jax
import jax.numpy as jnp
from jax import lax
from jax.experimental import pallas as pl
from jax.experimental.pallas import tpu as pltpu
from jax.experimental.pallas import tpu_sc as plsc

_D = 128
_BLK = 1000
_N = 10000
_M = 320000
_R = _N + _M
_W = 4000
_B = 64
_NB = 64
_RING = _NB * _B
_RB = 4
_CHL = 8000
_ACC_ROWS = _CHL + 8

_i32 = jnp.int32
_f32 = jnp.float32


def _zero16(ref, n):
    z = jnp.zeros((16,), ref.dtype)

    def body(j, _):
        ref[pl.ds(j * 16, 16)] = z
        return 0

    lax.fori_loop(0, n // 16, body, 0)


def _emit_subpass(spec, cid, sid, src_ref, esrc_ref, edst_ref, out_ref,
                  deg_out, zblk_ref, scr):
    (E, CH, nch, dst_lo0, out_base0, src_off, do_deg,
     deg_stripe, n_deg_sub, out_stripe, n_out_sub) = spec
    (dstbuf, srcbuf, cidx, gidx0, gidx1, gidx2, gidx3, locb0, locb1,
     locb2, locb3, rows0, rows1, rows2, rows3, onevec,
     zerovec, outst0, outst1, degstage, acc, degacc,
     wsem, gsem, zsem, osem, ssem0, ssem1, ssem2, ssem3) = scr
    rows = (rows0, rows1, rows2, rows3)
    gidx = (gidx0, gidx1, gidx2, gidx3)
    locb = (locb0, locb1, locb2, locb3)
    ssems = (ssem0, ssem1, ssem2, ssem3)
    e_per = E // 16
    nwin = e_per // _W
    n_my = (nch - cid + 1) // 2
    nz = out_stripe // 40

    def wload(w, slot):
        off = sid * e_per + w * _W
        pltpu.async_copy(edst_ref.at[pl.ds(off, _W)],
                         dstbuf.at[pl.ds(slot * _W, _W)], wsem)
        pltpu.async_copy(esrc_ref.at[pl.ds(off, _W)],
                         srcbuf.at[pl.ds(slot * _W, _W)], wsem)

    def wwait(w, slot):
        off = sid * e_per + w * _W
        pltpu.make_async_copy(edst_ref.at[pl.ds(off, _W)],
                              dstbuf.at[pl.ds(slot * _W, _W)], wsem).wait()
        pltpu.make_async_copy(esrc_ref.at[pl.ds(off, _W)],
                              srcbuf.at[pl.ds(slot * _W, _W)], wsem).wait()

    def fire(f):
        off = (f & (_NB - 1)) * _B
        sl = f & (_RB - 1)
        for si in range(_RB):
            @pl.when(sl == si)
            def _(si=si):
                @pl.when(f >= _RB)
                def _():
                    pltpu.make_async_copy(
                        rows[si], acc.at[locb[0]], ssems[si]).wait()
                for t in range(_B // 16):
                    v = cidx[pl.ds(off + t * 16, 16)]
                    gidx[si][pl.ds(t * 16, 16)] = v & 0x7FFFF
                    locb[si][pl.ds(t * 16, 16)] = (
                        lax.shift_right_logical(v, 19))
                pltpu.async_copy(src_ref.at[gidx[si]], rows[si], gsem)

    def gwait_any():
        pltpu.make_async_copy(
            src_ref.at[gidx[0]], rows0, gsem).wait()

    def scat(i):
        sl = i & (_RB - 1)
        for si in range(_RB):
            @pl.when(sl == si)
            def _(si=si):
                pltpu.async_copy(rows[si], acc.at[locb[si]], ssems[si],
                                 add=True)
                if do_deg:
                    pltpu.sync_copy(onevec, degacc.at[locb[si]], add=True)

    def chunk_body(k, _):
        c = cid + 2 * k
        dlo = dst_lo0 + c * CH
        obase = out_base0 + c * CH

        @pl.when(sid < n_out_sub)
        def _():
            pltpu.sync_copy(zblk_ref, outst0)

            def zi(b, _):
                pltpu.async_copy(
                    outst0, acc.at[pl.ds(sid * out_stripe + b * 40, 40)],
                    zsem)
                return 0

            lax.fori_loop(0, nz, zi, 0)

            def zw(b, _):
                pltpu.make_async_copy(
                    outst0, acc.at[pl.ds(sid * out_stripe + b * 40, 40)],
                    zsem).wait()
                return 0

            lax.fori_loop(0, nz, zw, 0)
        if do_deg:
            @pl.when(sid < n_deg_sub)
            def _():
                pltpu.sync_copy(
                    zerovec.at[pl.ds(0, deg_stripe)],
                    degacc.at[pl.ds(sid * deg_stripe, deg_stripe)])
        plsc.subcore_barrier()

        wload(0, 0)

        def win_body(w, carry):
            cc, ff = carry
            slot = lax.rem(w, 2)
            sbase = slot * _W
            wwait(w, slot)

            @pl.when(w + 1 < nwin)
            def _():
                wload(w + 1, 1 - slot)

            def filt(j, cnt_vec):
                d = dstbuf[pl.ds(sbase + j * 16, 16)]
                s = srcbuf[pl.ds(sbase + j * 16, 16)]
                m = (d >= dlo) & (d < dlo + CH)
                prefix = plsc.cumsum(jnp.where(m, _i32(1), _i32(0)))
                pos = cnt_vec + prefix - 1
                packed = lax.shift_left(d - dlo, 19) | (s + src_off)
                plsc.store_scatter(cidx, [pos & (_RING - 1)], packed,
                                   mask=m)
                return cnt_vec + plsc.all_reduce_population_count(m)

            cnt_vec = lax.fori_loop(0, _W // 16, filt,
                                    jnp.zeros((16,), _i32) + cc, unroll=8)
            cc2 = jnp.max(cnt_vec)

            def fcond(f):
                return (f + 1) * _B <= cc2

            def fbody(f):
                @pl.when(f >= 2)
                def _():
                    gwait_any()
                    scat(f - 2)

                fire(f)
                return f + 1

            ff = lax.while_loop(fcond, fbody, ff)
            return cc2, ff

        cc, ff = lax.fori_loop(
            0, nwin, win_body, (_i32(0), _i32(0)))

        cpad = (cc + _B - 1) // _B * _B
        dummy = jnp.full((16,), CH << 19, _i32)

        def padb(j, _):
            lane = lax.broadcasted_iota(_i32, (16,), 0) + j * 16
            minv = jnp.logical_not(lane < cc)
            plsc.store_scatter(cidx, [lane & (_RING - 1)], dummy, mask=minv)
            return 0

        lax.fori_loop(cc // 16, cpad // 16, padb, 0)

        def lcond(f):
            return f * _B < cpad

        def lbody(f):
            @pl.when(f >= 2)
            def _():
                gwait_any()
                scat(f - 2)

            fire(f)
            return f + 1

        ff = lax.while_loop(lcond, lbody, ff)

        def dcond(i):
            return i < ff

        def dbody(i):
            gwait_any()
            scat(i)
            return i + 1

        lax.while_loop(dcond, dbody, jnp.maximum(ff - 2, 0))

        nlast = jnp.minimum(ff, _RB)
        for si in range(_RB):
            @pl.when(si < nlast)
            def _(si=si):
                pltpu.make_async_copy(
                    rows[si], acc.at[locb[si]], ssems[si]).wait()
        plsc.subcore_barrier()

        @pl.when(sid < n_out_sub)
        def _():
            def ob(b, _):
                par = lax.rem(b, 2)
                roff = sid * out_stripe + b * 40

                @pl.when(par == 0)
                def _():
                    @pl.when(b >= 2)
                    def _():
                        pltpu.make_async_copy(
                            outst0, out_ref.at[pl.ds(0, 40)], osem).wait()
                    pltpu.sync_copy(acc.at[pl.ds(roff, 40)], outst0)
                    pltpu.async_copy(
                        outst0, out_ref.at[pl.ds(obase + roff, 40)], osem)

                @pl.when(par == 1)
                def _():
                    @pl.when(b >= 2)
                    def _():
                        pltpu.make_async_copy(
                            outst1, out_ref.at[pl.ds(0, 40)], osem).wait()
                    pltpu.sync_copy(acc.at[pl.ds(roff, 40)], outst1)
                    pltpu.async_copy(
                        outst1, out_ref.at[pl.ds(obase + roff, 40)], osem)
                return 0

            lax.fori_loop(0, nz, ob, 0)
            pltpu.make_async_copy(outst0, out_ref.at[pl.ds(0, 40)],
                                  osem).wait()
            pltpu.make_async_copy(outst0, out_ref.at[pl.ds(0, 40)],
                                  osem).wait()
        if do_deg:
            @pl.when(sid < n_deg_sub)
            def _():
                pltpu.sync_copy(
                    degacc.at[pl.ds(sid * deg_stripe, deg_stripe)],
                    degstage.at[pl.ds(0, deg_stripe)])
                pltpu.sync_copy(
                    degstage.at[pl.ds(0, deg_stripe)],
                    deg_out.at[pl.ds(obase + sid * deg_stripe, deg_stripe)])
        plsc.subcore_barrier()
        return 0

    lax.fori_loop(0, n_my, chunk_body, 0)


_SPEC_G = (320000, 5000, 2, 0, 0, 0, False, 1000, 5, 1000, 5)
_SPEC_LG = (2560000, _CHL, 40, 0, _N, 0, False, 800, 10, 800, 10)
_SPEC_G_DEG = (320000, 5000, 2, 0, 0, 0, True, 1000, 5, 1000, 5)
_SPEC_LG_DEG = (2560000, _CHL, 40, 0, _N, _N, True, 800, 10, 800, 10)
_SPEC_GLG_A = (1280000, 5000, 2, 0, 0, 0, False, 1000, 5, 1000, 5)
_SPEC_GLG_B = (1280000, _CHL, 40, _N, _N, 0, False, 800, 10, 800, 10)


def _init_const_bufs(onevec, zerovec):
    _zero16(zerovec, 1008)

    def ob(j, _):
        onevec[pl.ds(j * 16, 16)] = jnp.ones((16,), _f32)
        return 0
    lax.fori_loop(0, _B // 16, ob, 0)


def _k1_body(zblk, xg_ref, xlg_ref, esg, edg, eslg, edlg, out_ref, *scr):
    cid = lax.axis_index("c")
    sid = lax.axis_index("s")
    _init_const_bufs(scr[15], scr[16])
    scr = list(scr[:21]) + [None] + list(scr[21:])
    _emit_subpass(_SPEC_LG, cid, sid, xlg_ref, eslg, edlg, out_ref, None,
                  zblk, scr)
    _emit_subpass(_SPEC_G, cid, sid, xg_ref, esg, edg, out_ref, None,
                  zblk, scr)


def _k2_body(zblk, z1_ref, esg, edg, eslg, edlg, out_ref, deg_ref, *scr):
    cid = lax.axis_index("c")
    sid = lax.axis_index("s")
    _init_const_bufs(scr[15], scr[16])
    _emit_subpass(_SPEC_LG_DEG, cid, sid, z1_ref, eslg, edlg, out_ref,
                  deg_ref, zblk, scr)
    _emit_subpass(_SPEC_G_DEG, cid, sid, z1_ref, esg, edg, out_ref,
                  deg_ref, zblk, scr)


def _k3_body(zblk, xf_ref, esglg, edglg, out_ref, *scr):
    cid = lax.axis_index("c")
    sid = lax.axis_index("s")
    _init_const_bufs(scr[15], scr[16])
    scr = list(scr[:21]) + [None] + list(scr[21:])
    _emit_subpass(_SPEC_GLG_B, cid, sid, xf_ref, esglg, edglg, out_ref,
                  None, zblk, scr)
    _emit_subpass(_SPEC_GLG_A, cid, sid, xf_ref, esglg, edglg, out_ref,
                  None, zblk, scr)


def _sc_scratch(with_deg):
    scr = [
        pltpu.VMEM((2 * _W,), _i32),
        pltpu.VMEM((2 * _W,), _i32),
        pltpu.VMEM((_RING,), _i32),
        pltpu.VMEM((_B,), _i32),
        pltpu.VMEM((_B,), _i32),
        pltpu.VMEM((_B,), _i32),
        pltpu.VMEM((_B,), _i32),
        pltpu.VMEM((_B,), _i32),
        pltpu.VMEM((_B,), _i32),
        pltpu.VMEM((_B,), _i32),
        pltpu.VMEM((_B,), _i32),
        pltpu.VMEM((_B, _D), _f32),
        pltpu.VMEM((_B, _D), _f32),
        pltpu.VMEM((_B, _D), _f32),
        pltpu.VMEM((_B, _D), _f32),
        pltpu.VMEM((_B,), _f32),
        pltpu.VMEM((1008,), _f32),
        pltpu.VMEM((40, _D), _f32),
        pltpu.VMEM((40, _D), _f32),
        pltpu.VMEM((1008,), _f32),
        pltpu.VMEM_SHARED((_ACC_ROWS, _D), _f32),
    ]
    if with_deg:
        scr.append(pltpu.VMEM_SHARED((_ACC_ROWS,), _f32))
    scr += [pltpu.SemaphoreType.DMA] * 8
    return scr


def _mesh():
    return plsc.VectorSubcoreMesh(core_axis_name="c", subcore_axis_name="s",
                                  num_cores=2, num_subcores=16)


_SC_PARAMS = pltpu.CompilerParams(needs_layout_passes=False)


def _glob_body(x_ref, o_ref):
    i = pl.program_id(0)

    @pl.when(i == 0)
    def _():
        o_ref[...] = jnp.zeros_like(o_ref)

    s = jnp.sum(x_ref[...], axis=0, keepdims=True)
    r = jnp.where(i < _N // _BLK, 0, 1)
    o_ref[pl.ds(r, 1), :] += s


def _glob_sums(xf):
    return pl.pallas_call(
        _glob_body,
        grid=(_R // _BLK,),
        in_specs=[pl.BlockSpec((_BLK, _D), lambda i: (i, 0))],
        out_specs=pl.BlockSpec((8, _D), lambda i: (0, 0)),
        out_shape=jax.ShapeDtypeStruct((8, _D), _f32),
    )(xf)


def _update_body(glob_ref, wcat_ref, w3_ref, ball_ref, xf_ref, y_ref, z1_ref,
                 deg_ref, out_ref):
    xf = xf_ref[...]
    cat = jnp.concatenate(
        [xf, y_ref[...], xf * deg_ref[...], z1_ref[...]], axis=1)
    acc = lax.dot_general(cat, wcat_ref[...], (((1,), (0,)), ((), ())),
                          preferred_element_type=_f32)
    cvec = lax.dot_general(glob_ref[...], w3_ref[...],
                           (((1,), (0,)), ((), ())),
                           preferred_element_type=_f32)
    out_ref[...] = acc + cvec + ball_ref[...]


def _update(xf, y, z1, deg, glob, wcat, w3, ball, row0, rows):
    blk0 = row0 // _BLK

    def rmap(i):
        return (i + blk0, 0)

    return pl.pallas_call(
        _update_body,
        grid=(rows // _BLK,),
        in_specs=[
            pl.BlockSpec((1, _D), lambda i: (0, 0)),
            pl.BlockSpec((4 * _D, _D), lambda i: (0, 0)),
            pl.BlockSpec((_D, _D), lambda i: (0, 0)),
            pl.BlockSpec((1, _D), lambda i: (0, 0)),
            pl.BlockSpec((_BLK, _D), rmap),
            pl.BlockSpec((_BLK, _D), rmap),
            pl.BlockSpec((_BLK, _D), rmap),
            pl.BlockSpec((_BLK, 1), rmap),
        ],
        out_specs=pl.BlockSpec((_BLK, _D), lambda i: (i, 0)),
        out_shape=jax.ShapeDtypeStruct((rows, _D), _f32),
    )(glob, wcat, w3, ball, xf, y, z1, deg)


def kernel(x_g, x_lg, edge_index_g, edge_index_lg, edge_index_glg,
           Wt_main, bt_main, Wt_list, bt_list,
           Wg_main, bg_main, Wg_list, bg_list):
    esg, edg = edge_index_g[0], edge_index_g[1]
    eslg, edlg = edge_index_lg[0], edge_index_lg[1]
    esglg, edglg = edge_index_glg[0], edge_index_glg[1]
    zblk = jnp.zeros((40, _D), _f32)

    z1 = pl.kernel(
        _k1_body,
        out_type=jax.ShapeDtypeStruct((_R, _D), _f32),
        mesh=_mesh(),
        scratch_types=_sc_scratch(False),
        compiler_params=_SC_PARAMS,
    )(zblk, x_g, x_lg, esg, edg, eslg, edlg)

    xf, deg = pl.kernel(
        _k2_body,
        out_type=(jax.ShapeDtypeStruct((_R, _D), _f32),
                  jax.ShapeDtypeStruct((_R,), _f32)),
        mesh=_mesh(),
        scratch_types=_sc_scratch(True),
        compiler_params=_SC_PARAMS,
    )(zblk, z1, esg, edg, eslg, edlg)

    y = pl.kernel(
        _k3_body,
        out_type=jax.ShapeDtypeStruct((_R, _D), _f32),
        mesh=_mesh(),
        scratch_types=_sc_scratch(False),
        compiler_params=_SC_PARAMS,
    )(zblk, xf, esglg, edglg)

    gs = _glob_sums(xf)
    glob_g = gs[0:1] / _N
    glob_lg = gs[1:2] / _M

    wcat_t = jnp.concatenate(
        [Wt_main[0] + Wt_list[1], Wt_main[1], Wt_main[2], Wt_list[0]], axis=0)
    ball_t = (bt_main.sum(0) + bt_list.sum(0))[None, :]
    wcat_g = jnp.concatenate(
        [Wg_main[0] + Wg_list[1], Wg_main[1], Wg_main[2], Wg_list[0]], axis=0)
    ball_g = (bg_main.sum(0) + bg_list.sum(0))[None, :]

    deg2 = deg[:, None]
    out_g = _update(xf, y, z1, deg2, glob_g, wcat_t, Wt_main[3], ball_t,
                    0, _N)
    out_lg = _update(xf, y, z1, deg2, glob_lg, wcat_g, Wg_main[3], ball_g,
                     _N, _M)
    return (out_g, out_lg)


# speedup vs baseline: 1.0204x; 1.0204x over previous
import functools

import jax
import jax.numpy as jnp
from jax import lax
from jax.experimental import pallas as pl
from jax.experimental.pallas import tpu as pltpu
from jax.experimental.pallas import tpu_sc as plsc

_D = 128
_BLK = 1000
_N = 10000
_M = 320000
_R = _N + _M
_W = 2000
_B = 64
_NB = 64
_RING = _NB * _B
_RB = 4
_CHL = 8000
_ACC_ROWS = _CHL + 8

_i32 = jnp.int32
_f32 = jnp.float32


def _zero16(ref, n):
    z = jnp.zeros((16,), ref.dtype)

    def body(j, _):
        ref[pl.ds(j * 16, 16)] = z
        return 0

    lax.fori_loop(0, n // 16, body, 0)


def _emit_subpass(spec, cid, sid, src_ref, esrc_ref, edst_ref, out_ref,
                  deg_out, zblk_ref, scr):
    (E, CH, nch, dst_lo0, out_base0, src_off, do_deg,
     deg_stripe, n_deg_sub, out_stripe, n_out_sub) = spec
    (dstbuf, srcbuf, cidx, gidx0, gidx1, gidx2, gidx3, locb0, locb1,
     locb2, locb3, rows0, rows1, rows2, rows3, onevec,
     zerovec, outst0, outst1, degstage, acc, degacc,
     wsem, gsem, zsem, osem, ssem0, ssem1, ssem2, ssem3) = scr
    rows = (rows0, rows1, rows2, rows3)
    gidx = (gidx0, gidx1, gidx2, gidx3)
    locb = (locb0, locb1, locb2, locb3)
    ssems = (ssem0, ssem1, ssem2, ssem3)
    e_per = E // 16
    nwin = e_per // _W
    n_my = (nch - cid + 1) // 2
    nz = out_stripe // 40

    def wload(w, slot):
        off = sid * e_per + w * _W
        pltpu.async_copy(edst_ref.at[pl.ds(off, _W)],
                         dstbuf.at[pl.ds(slot * _W, _W)], wsem)
        pltpu.async_copy(esrc_ref.at[pl.ds(off, _W)],
                         srcbuf.at[pl.ds(slot * _W, _W)], wsem)

    def wwait(w, slot):
        off = sid * e_per + w * _W
        pltpu.make_async_copy(edst_ref.at[pl.ds(off, _W)],
                              dstbuf.at[pl.ds(slot * _W, _W)], wsem).wait()
        pltpu.make_async_copy(esrc_ref.at[pl.ds(off, _W)],
                              srcbuf.at[pl.ds(slot * _W, _W)], wsem).wait()

    def fire(f):
        off = (f & (_NB - 1)) * _B
        sl = f & (_RB - 1)
        for si in range(_RB):
            @pl.when(sl == si)
            def _(si=si):
                @pl.when(f >= _RB)
                def _():
                    pltpu.make_async_copy(
                        rows[si], acc.at[locb[0]], ssems[si]).wait()
                for t in range(_B // 16):
                    v = cidx[pl.ds(off + t * 16, 16)]
                    gidx[si][pl.ds(t * 16, 16)] = v & 0x7FFFF
                    locb[si][pl.ds(t * 16, 16)] = (
                        lax.shift_right_logical(v, 19))
                pltpu.async_copy(src_ref.at[gidx[si]], rows[si], gsem)

    def gwait_any():
        pltpu.make_async_copy(
            src_ref.at[gidx[0]], rows0, gsem).wait()

    def scat(i):
        sl = i & (_RB - 1)
        for si in range(_RB):
            @pl.when(sl == si)
            def _(si=si):
                pltpu.async_copy(rows[si], acc.at[locb[si]], ssems[si],
                                 add=True)
                if do_deg:
                    pltpu.sync_copy(onevec, degacc.at[locb[si]], add=True)

    def chunk_body(k, _):
        c = cid + 2 * k
        dlo = dst_lo0 + c * CH
        obase = out_base0 + c * CH

        @pl.when(sid < n_out_sub)
        def _():
            pltpu.sync_copy(zblk_ref, outst0)

            def zi(b, _):
                pltpu.async_copy(
                    outst0, acc.at[pl.ds(sid * out_stripe + b * 40, 40)],
                    zsem)
                return 0

            lax.fori_loop(0, nz, zi, 0)

            def zw(b, _):
                pltpu.make_async_copy(
                    outst0, acc.at[pl.ds(sid * out_stripe + b * 40, 40)],
                    zsem).wait()
                return 0

            lax.fori_loop(0, nz, zw, 0)
        if do_deg:
            @pl.when(sid < n_deg_sub)
            def _():
                pltpu.sync_copy(
                    zerovec.at[pl.ds(0, deg_stripe)],
                    degacc.at[pl.ds(sid * deg_stripe, deg_stripe)])
        plsc.subcore_barrier()

        wload(0, 0)

        def win_body(w, carry):
            cc, ff = carry
            slot = lax.rem(w, 2)
            sbase = slot * _W
            wwait(w, slot)

            @pl.when(w + 1 < nwin)
            def _():
                wload(w + 1, 1 - slot)

            def filt(j, cnt_vec):
                d = dstbuf[pl.ds(sbase + j * 16, 16)]
                s = srcbuf[pl.ds(sbase + j * 16, 16)]
                m = (d >= dlo) & (d < dlo + CH)
                prefix = plsc.cumsum(jnp.where(m, _i32(1), _i32(0)))
                pos = cnt_vec + prefix - 1
                packed = lax.shift_left(d - dlo, 19) | (s + src_off)
                plsc.store_scatter(cidx, [pos & (_RING - 1)], packed,
                                   mask=m)
                return cnt_vec + plsc.all_reduce_population_count(m)

            cnt_vec = lax.fori_loop(0, _W // 16, filt,
                                    jnp.zeros((16,), _i32) + cc, unroll=16)
            cc2 = jnp.max(cnt_vec)

            def fcond(f):
                return (f + 1) * _B <= cc2

            def fbody(f):
                @pl.when(f >= 2)
                def _():
                    gwait_any()
                    scat(f - 2)

                fire(f)
                return f + 1

            ff = lax.while_loop(fcond, fbody, ff)
            return cc2, ff

        cc, ff = lax.fori_loop(
            0, nwin, win_body, (_i32(0), _i32(0)))

        cpad = (cc + _B - 1) // _B * _B
        dummy = jnp.full((16,), CH << 19, _i32)

        def padb(j, _):
            lane = lax.broadcasted_iota(_i32, (16,), 0) + j * 16
            minv = jnp.logical_not(lane < cc)
            plsc.store_scatter(cidx, [lane & (_RING - 1)], dummy, mask=minv)
            return 0

        lax.fori_loop(cc // 16, cpad // 16, padb, 0)

        def lcond(f):
            return f * _B < cpad

        def lbody(f):
            @pl.when(f >= 2)
            def _():
                gwait_any()
                scat(f - 2)

            fire(f)
            return f + 1

        ff = lax.while_loop(lcond, lbody, ff)

        def dcond(i):
            return i < ff

        def dbody(i):
            gwait_any()
            scat(i)
            return i + 1

        lax.while_loop(dcond, dbody, jnp.maximum(ff - 2, 0))

        nlast = jnp.minimum(ff, _RB)
        for si in range(_RB):
            @pl.when(si < nlast)
            def _(si=si):
                pltpu.make_async_copy(
                    rows[si], acc.at[locb[si]], ssems[si]).wait()
        plsc.subcore_barrier()

        @pl.when(sid < n_out_sub)
        def _():
            def ob(b, _):
                par = lax.rem(b, 2)
                roff = sid * out_stripe + b * 40

                @pl.when(par == 0)
                def _():
                    @pl.when(b >= 2)
                    def _():
                        pltpu.make_async_copy(
                            outst0, out_ref.at[pl.ds(0, 40)], osem).wait()
                    pltpu.sync_copy(acc.at[pl.ds(roff, 40)], outst0)
                    pltpu.async_copy(
                        outst0, out_ref.at[pl.ds(obase + roff, 40)], osem)

                @pl.when(par == 1)
                def _():
                    @pl.when(b >= 2)
                    def _():
                        pltpu.make_async_copy(
                            outst1, out_ref.at[pl.ds(0, 40)], osem).wait()
                    pltpu.sync_copy(acc.at[pl.ds(roff, 40)], outst1)
                    pltpu.async_copy(
                        outst1, out_ref.at[pl.ds(obase + roff, 40)], osem)
                return 0

            lax.fori_loop(0, nz, ob, 0)
            pltpu.make_async_copy(outst0, out_ref.at[pl.ds(0, 40)],
                                  osem).wait()
            pltpu.make_async_copy(outst0, out_ref.at[pl.ds(0, 40)],
                                  osem).wait()
        if do_deg:
            @pl.when(sid < n_deg_sub)
            def _():
                pltpu.sync_copy(
                    degacc.at[pl.ds(sid * deg_stripe, deg_stripe)],
                    degstage.at[pl.ds(0, deg_stripe)])
                pltpu.sync_copy(
                    degstage.at[pl.ds(0, deg_stripe)],
                    deg_out.at[pl.ds(obase + sid * deg_stripe, deg_stripe)])
        plsc.subcore_barrier()
        return 0

    lax.fori_loop(0, n_my, chunk_body, 0)


_SPEC_G = (320000, 5000, 2, 0, 0, 0, False, 1000, 5, 1000, 5)
_SPEC_LG = (2560000, _CHL, 40, 0, _N, 0, False, 800, 10, 800, 10)
_SPEC_G_DEG = (320000, 5000, 2, 0, 0, 0, True, 1000, 5, 1000, 5)
_SPEC_LG_DEG = (2560000, _CHL, 40, 0, _N, _N, True, 800, 10, 800, 10)
_SPEC_GLG_A = (1280000, 5000, 2, 0, 0, 0, False, 1000, 5, 1000, 5)
_SPEC_GLG_B = (1280000, _CHL, 40, _N, _N, 0, False, 800, 10, 800, 10)


def _init_const_bufs(onevec, zerovec):
    _zero16(zerovec, 1008)

    def ob(j, _):
        onevec[pl.ds(j * 16, 16)] = jnp.ones((16,), _f32)
        return 0
    lax.fori_loop(0, _B // 16, ob, 0)


def _k1_body(zblk, xg_ref, xlg_ref, esg, edg, eslg, edlg, out_ref, *scr):
    cid = lax.axis_index("c")
    sid = lax.axis_index("s")
    _init_const_bufs(scr[15], scr[16])
    scr = list(scr[:21]) + [None] + list(scr[21:])
    _emit_subpass(_SPEC_LG, cid, sid, xlg_ref, eslg, edlg, out_ref, None,
                  zblk, scr)
    _emit_subpass(_SPEC_G, cid, sid, xg_ref, esg, edg, out_ref, None,
                  zblk, scr)


def _k2_body(zblk, z1_ref, esg, edg, eslg, edlg, out_ref, deg_ref, *scr):
    cid = lax.axis_index("c")
    sid = lax.axis_index("s")
    _init_const_bufs(scr[15], scr[16])
    _emit_subpass(_SPEC_LG_DEG, cid, sid, z1_ref, eslg, edlg, out_ref,
                  deg_ref, zblk, scr)
    _emit_subpass(_SPEC_G_DEG, cid, sid, z1_ref, esg, edg, out_ref,
                  deg_ref, zblk, scr)


def _k3_body(zblk, xf_ref, esglg, edglg, out_ref, *scr):
    cid = lax.axis_index("c")
    sid = lax.axis_index("s")
    _init_const_bufs(scr[15], scr[16])
    scr = list(scr[:21]) + [None] + list(scr[21:])
    _emit_subpass(_SPEC_GLG_B, cid, sid, xf_ref, esglg, edglg, out_ref,
                  None, zblk, scr)
    _emit_subpass(_SPEC_GLG_A, cid, sid, xf_ref, esglg, edglg, out_ref,
                  None, zblk, scr)


def _sc_scratch(with_deg):
    scr = [
        pltpu.VMEM((2 * _W,), _i32),
        pltpu.VMEM((2 * _W,), _i32),
        pltpu.VMEM((_RING,), _i32),
        pltpu.VMEM((_B,), _i32),
        pltpu.VMEM((_B,), _i32),
        pltpu.VMEM((_B,), _i32),
        pltpu.VMEM((_B,), _i32),
        pltpu.VMEM((_B,), _i32),
        pltpu.VMEM((_B,), _i32),
        pltpu.VMEM((_B,), _i32),
        pltpu.VMEM((_B,), _i32),
        pltpu.VMEM((_B, _D), _f32),
        pltpu.VMEM((_B, _D), _f32),
        pltpu.VMEM((_B, _D), _f32),
        pltpu.VMEM((_B, _D), _f32),
        pltpu.VMEM((_B,), _f32),
        pltpu.VMEM((1008,), _f32),
        pltpu.VMEM((40, _D), _f32),
        pltpu.VMEM((40, _D), _f32),
        pltpu.VMEM((1008,), _f32),
        pltpu.VMEM_SHARED((_ACC_ROWS, _D), _f32),
    ]
    if with_deg:
        scr.append(pltpu.VMEM_SHARED((_ACC_ROWS,), _f32))
    scr += [pltpu.SemaphoreType.DMA] * 8
    return scr


def _mesh():
    return plsc.VectorSubcoreMesh(core_axis_name="c", subcore_axis_name="s",
                                  num_cores=2, num_subcores=16)


_SC_PARAMS = pltpu.CompilerParams(needs_layout_passes=False)


def _glob_body(x_ref, o_ref):
    i = pl.program_id(0)

    @pl.when(i == 0)
    def _():
        o_ref[...] = jnp.zeros_like(o_ref)

    s = jnp.sum(x_ref[...], axis=0, keepdims=True)
    r = jnp.where(i < _N // _BLK, 0, 1)
    o_ref[pl.ds(r, 1), :] += s


def _glob_sums(xf):
    return pl.pallas_call(
        _glob_body,
        grid=(_R // _BLK,),
        in_specs=[pl.BlockSpec((_BLK, _D), lambda i: (i, 0))],
        out_specs=pl.BlockSpec((8, _D), lambda i: (0, 0)),
        out_shape=jax.ShapeDtypeStruct((8, _D), _f32),
    )(xf)


def _update_body(glob_ref, wcat_ref, w3_ref, ball_ref, xf_ref, y_ref, z1_ref,
                 deg_ref, out_ref):
    xf = xf_ref[...]
    cat = jnp.concatenate(
        [xf, y_ref[...], xf * deg_ref[...], z1_ref[...]], axis=1)
    acc = lax.dot_general(cat, wcat_ref[...], (((1,), (0,)), ((), ())),
                          preferred_element_type=_f32)
    cvec = lax.dot_general(glob_ref[...], w3_ref[...],
                           (((1,), (0,)), ((), ())),
                           preferred_element_type=_f32)
    out_ref[...] = acc + cvec + ball_ref[...]


def _update(xf, y, z1, deg, glob, wcat, w3, ball, row0, rows):
    blk0 = row0 // _BLK

    def rmap(i):
        return (i + blk0, 0)

    return pl.pallas_call(
        _update_body,
        grid=(rows // _BLK,),
        in_specs=[
            pl.BlockSpec((1, _D), lambda i: (0, 0)),
            pl.BlockSpec((4 * _D, _D), lambda i: (0, 0)),
            pl.BlockSpec((_D, _D), lambda i: (0, 0)),
            pl.BlockSpec((1, _D), lambda i: (0, 0)),
            pl.BlockSpec((_BLK, _D), rmap),
            pl.BlockSpec((_BLK, _D), rmap),
            pl.BlockSpec((_BLK, _D), rmap),
            pl.BlockSpec((_BLK, 1), rmap),
        ],
        out_specs=pl.BlockSpec((_BLK, _D), lambda i: (i, 0)),
        out_shape=jax.ShapeDtypeStruct((rows, _D), _f32),
    )(glob, wcat, w3, ball, xf, y, z1, deg)


def kernel(x_g, x_lg, edge_index_g, edge_index_lg, edge_index_glg,
           Wt_main, bt_main, Wt_list, bt_list,
           Wg_main, bg_main, Wg_list, bg_list):
    esg, edg = edge_index_g[0], edge_index_g[1]
    eslg, edlg = edge_index_lg[0], edge_index_lg[1]
    esglg, edglg = edge_index_glg[0], edge_index_glg[1]
    zblk = jnp.zeros((40, _D), _f32)

    z1 = pl.kernel(
        _k1_body,
        out_type=jax.ShapeDtypeStruct((_R, _D), _f32),
        mesh=_mesh(),
        scratch_types=_sc_scratch(False),
        compiler_params=_SC_PARAMS,
    )(zblk, x_g, x_lg, esg, edg, eslg, edlg)

    xf, deg = pl.kernel(
        _k2_body,
        out_type=(jax.ShapeDtypeStruct((_R, _D), _f32),
                  jax.ShapeDtypeStruct((_R,), _f32)),
        mesh=_mesh(),
        scratch_types=_sc_scratch(True),
        compiler_params=_SC_PARAMS,
    )(zblk, z1, esg, edg, eslg, edlg)

    y = pl.kernel(
        _k3_body,
        out_type=jax.ShapeDtypeStruct((_R, _D), _f32),
        mesh=_mesh(),
        scratch_types=_sc_scratch(False),
        compiler_params=_SC_PARAMS,
    )(zblk, xf, esglg, edglg)

    gs = _glob_sums(xf)
    glob_g = gs[0:1] / _N
    glob_lg = gs[1:2] / _M

    wcat_t = jnp.concatenate(
        [Wt_main[0] + Wt_list[1], Wt_main[1], Wt_main[2], Wt_list[0]], axis=0)
    ball_t = (bt_main.sum(0) + bt_list.sum(0))[None, :]
    wcat_g = jnp.concatenate(
        [Wg_main[0] + Wg_list[1], Wg_main[1], Wg_main[2], Wg_list[0]], axis=0)
    ball_g = (bg_main.sum(0) + bg_list.sum(0))[None, :]

    deg2 = deg[:, None]
    out_g = _update(xf, y, z1, deg2, glob_g, wcat_t, Wt_main[3], ball_t,
                    0, _N)
    out_lg = _update(xf, y, z1, deg2, glob_lg, wcat_g, Wg_main[3], ball_g,
                     _N, _M)
    return (out_g, out_lg)


# speedup vs baseline: 1.0221x; 1.0017x over previous
import functools

import jax
import jax.numpy as jnp
from jax import lax
from jax.experimental import pallas as pl
from jax.experimental.pallas import tpu as pltpu
from jax.experimental.pallas import tpu_sc as plsc

_D = 128
_BLK = 1000
_N = 10000
_M = 320000
_R = _N + _M
_W = 2000
_B = 64
_NB = 64
_RING = _NB * _B
_RB = 4
_CHL = 8000
_ACC_ROWS = _CHL + 8

_i32 = jnp.int32
_f32 = jnp.float32


def _zero16(ref, n):
    z = jnp.zeros((16,), ref.dtype)

    def body(j, _):
        ref[pl.ds(j * 16, 16)] = z
        return 0

    lax.fori_loop(0, n // 16, body, 0)


def _emit_subpass(spec, cid, sid, src_ref, esrc_ref, edst_ref, out_ref,
                  deg_out, zblk_ref, scr):
    (E, CH, nch, dst_lo0, out_base0, src_off, do_deg,
     deg_stripe, n_deg_sub, out_stripe, n_out_sub) = spec
    (dstbuf, srcbuf, cidx, gidx0, gidx1, gidx2, gidx3, locb0, locb1,
     locb2, locb3, rows0, rows1, rows2, rows3, onevec,
     zerovec, outst0, outst1, degstage, acc, degacc,
     wsem, gsem, zsem, osem, ssem0, ssem1, ssem2, ssem3) = scr
    rows = (rows0, rows1, rows2, rows3)
    gidx = (gidx0, gidx1, gidx2, gidx3)
    locb = (locb0, locb1, locb2, locb3)
    ssems = (ssem0, ssem1, ssem2, ssem3)
    e_per = E // 16
    nwin = e_per // _W
    n_my = (nch - cid + 1) // 2
    nz = out_stripe // 40

    def wload(w, slot):
        off = sid * e_per + w * _W
        pltpu.async_copy(edst_ref.at[pl.ds(off, _W)],
                         dstbuf.at[pl.ds(slot * _W, _W)], wsem)
        pltpu.async_copy(esrc_ref.at[pl.ds(off, _W)],
                         srcbuf.at[pl.ds(slot * _W, _W)], wsem)

    def wwait(w, slot):
        off = sid * e_per + w * _W
        pltpu.make_async_copy(edst_ref.at[pl.ds(off, _W)],
                              dstbuf.at[pl.ds(slot * _W, _W)], wsem).wait()
        pltpu.make_async_copy(esrc_ref.at[pl.ds(off, _W)],
                              srcbuf.at[pl.ds(slot * _W, _W)], wsem).wait()

    def fire(f):
        off = (f & (_NB - 1)) * _B
        sl = f & (_RB - 1)
        for si in range(_RB):
            @pl.when(sl == si)
            def _(si=si):
                @pl.when(f >= _RB)
                def _():
                    pltpu.make_async_copy(
                        rows[si], acc.at[locb[0]], ssems[si]).wait()
                for t in range(_B // 16):
                    v = cidx[pl.ds(off + t * 16, 16)]
                    gidx[si][pl.ds(t * 16, 16)] = v & 0x7FFFF
                    locb[si][pl.ds(t * 16, 16)] = (
                        lax.shift_right_logical(v, 19))
                pltpu.async_copy(src_ref.at[gidx[si]], rows[si], gsem)

    def gwait_any():
        pltpu.make_async_copy(
            src_ref.at[gidx[0]], rows0, gsem).wait()

    def scat(i):
        sl = i & (_RB - 1)
        for si in range(_RB):
            @pl.when(sl == si)
            def _(si=si):
                pltpu.async_copy(rows[si], acc.at[locb[si]], ssems[si],
                                 add=True)
                if do_deg:
                    pltpu.sync_copy(onevec, degacc.at[locb[si]], add=True)

    def chunk_body(k, _):
        c = cid + 2 * k
        dlo = dst_lo0 + c * CH
        obase = out_base0 + c * CH

        @pl.when(sid < n_out_sub)
        def _():
            pltpu.sync_copy(zblk_ref, outst0)

            def zi(b, _):
                pltpu.async_copy(
                    outst0, acc.at[pl.ds(sid * out_stripe + b * 40, 40)],
                    zsem)
                return 0

            lax.fori_loop(0, nz, zi, 0)

            def zw(b, _):
                pltpu.make_async_copy(
                    outst0, acc.at[pl.ds(sid * out_stripe + b * 40, 40)],
                    zsem).wait()
                return 0

            lax.fori_loop(0, nz, zw, 0)
        if do_deg:
            @pl.when(sid < n_deg_sub)
            def _():
                pltpu.sync_copy(
                    zerovec.at[pl.ds(0, deg_stripe)],
                    degacc.at[pl.ds(sid * deg_stripe, deg_stripe)])
        plsc.subcore_barrier()

        wload(0, 0)

        def win_body(w, carry):
            cc, ff = carry
            slot = lax.rem(w, 2)
            sbase = slot * _W
            wwait(w, slot)

            @pl.when(w + 1 < nwin)
            def _():
                wload(w + 1, 1 - slot)

            def filt(j, cnt_vec):
                d = dstbuf[pl.ds(sbase + j * 16, 16)]
                s = srcbuf[pl.ds(sbase + j * 16, 16)]
                m = (d >= dlo) & (d < dlo + CH)
                prefix = plsc.cumsum(jnp.where(m, _i32(1), _i32(0)))
                pos = cnt_vec + prefix - 1
                sv = s + src_off if src_off else s
                packed = lax.shift_left(d - dlo, 19) | sv
                plsc.store_scatter(cidx, [pos & (_RING - 1)], packed,
                                   mask=m)
                return cnt_vec + plsc.all_reduce_population_count(m)

            cnt_vec = lax.fori_loop(0, _W // 16, filt,
                                    jnp.zeros((16,), _i32) + cc, unroll=8)
            cc2 = jnp.max(cnt_vec)

            def fcond(f):
                return (f + 1) * _B <= cc2

            def fbody(f):
                @pl.when(f >= 2)
                def _():
                    gwait_any()
                    scat(f - 2)

                fire(f)
                return f + 1

            ff = lax.while_loop(fcond, fbody, ff)
            return cc2, ff

        cc, ff = lax.fori_loop(
            0, nwin, win_body, (_i32(0), _i32(0)))

        cpad = (cc + _B - 1) // _B * _B
        dummy = jnp.full((16,), CH << 19, _i32)

        def padb(j, _):
            lane = lax.broadcasted_iota(_i32, (16,), 0) + j * 16
            minv = jnp.logical_not(lane < cc)
            plsc.store_scatter(cidx, [lane & (_RING - 1)], dummy, mask=minv)
            return 0

        lax.fori_loop(cc // 16, cpad // 16, padb, 0)

        def lcond(f):
            return f * _B < cpad

        def lbody(f):
            @pl.when(f >= 2)
            def _():
                gwait_any()
                scat(f - 2)

            fire(f)
            return f + 1

        ff = lax.while_loop(lcond, lbody, ff)

        def dcond(i):
            return i < ff

        def dbody(i):
            gwait_any()
            scat(i)
            return i + 1

        lax.while_loop(dcond, dbody, jnp.maximum(ff - 2, 0))

        nlast = jnp.minimum(ff, _RB)
        for si in range(_RB):
            @pl.when(si < nlast)
            def _(si=si):
                pltpu.make_async_copy(
                    rows[si], acc.at[locb[si]], ssems[si]).wait()
        plsc.subcore_barrier()

        @pl.when(sid < n_out_sub)
        def _():
            def ob(b, _):
                par = lax.rem(b, 2)
                roff = sid * out_stripe + b * 40

                @pl.when(par == 0)
                def _():
                    @pl.when(b >= 2)
                    def _():
                        pltpu.make_async_copy(
                            outst0, out_ref.at[pl.ds(0, 40)], osem).wait()
                    pltpu.sync_copy(acc.at[pl.ds(roff, 40)], outst0)
                    pltpu.async_copy(
                        outst0, out_ref.at[pl.ds(obase + roff, 40)], osem)

                @pl.when(par == 1)
                def _():
                    @pl.when(b >= 2)
                    def _():
                        pltpu.make_async_copy(
                            outst1, out_ref.at[pl.ds(0, 40)], osem).wait()
                    pltpu.sync_copy(acc.at[pl.ds(roff, 40)], outst1)
                    pltpu.async_copy(
                        outst1, out_ref.at[pl.ds(obase + roff, 40)], osem)
                return 0

            lax.fori_loop(0, nz, ob, 0)
            pltpu.make_async_copy(outst0, out_ref.at[pl.ds(0, 40)],
                                  osem).wait()
            pltpu.make_async_copy(outst0, out_ref.at[pl.ds(0, 40)],
                                  osem).wait()
        if do_deg:
            @pl.when(sid < n_deg_sub)
            def _():
                pltpu.sync_copy(
                    degacc.at[pl.ds(sid * deg_stripe, deg_stripe)],
                    degstage.at[pl.ds(0, deg_stripe)])
                pltpu.sync_copy(
                    degstage.at[pl.ds(0, deg_stripe)],
                    deg_out.at[pl.ds(obase + sid * deg_stripe, deg_stripe)])
        plsc.subcore_barrier()
        return 0

    lax.fori_loop(0, n_my, chunk_body, 0)


_SPEC_G = (320000, 5000, 2, 0, 0, 0, False, 1000, 5, 1000, 5)
_SPEC_LG = (2560000, _CHL, 40, 0, _N, 0, False, 800, 10, 800, 10)
_SPEC_G_DEG = (320000, 5000, 2, 0, 0, 0, True, 1000, 5, 1000, 5)
_SPEC_LG_DEG = (2560000, _CHL, 40, 0, _N, _N, True, 800, 10, 800, 10)
_SPEC_GLG_A = (1280000, 5000, 2, 0, 0, 0, False, 1000, 5, 1000, 5)
_SPEC_GLG_B = (1280000, _CHL, 40, _N, _N, 0, False, 800, 10, 800, 10)


def _init_const_bufs(onevec, zerovec):
    _zero16(zerovec, 1008)

    def ob(j, _):
        onevec[pl.ds(j * 16, 16)] = jnp.ones((16,), _f32)
        return 0
    lax.fori_loop(0, _B // 16, ob, 0)


def _k1_body(zblk, xg_ref, xlg_ref, esg, edg, eslg, edlg, out_ref, *scr):
    cid = lax.axis_index("c")
    sid = lax.axis_index("s")
    _init_const_bufs(scr[15], scr[16])
    scr = list(scr[:21]) + [None] + list(scr[21:])
    _emit_subpass(_SPEC_LG, cid, sid, xlg_ref, eslg, edlg, out_ref, None,
                  zblk, scr)
    _emit_subpass(_SPEC_G, cid, sid, xg_ref, esg, edg, out_ref, None,
                  zblk, scr)


def _k2_body(zblk, z1_ref, esg, edg, eslg, edlg, out_ref, deg_ref, *scr):
    cid = lax.axis_index("c")
    sid = lax.axis_index("s")
    _init_const_bufs(scr[15], scr[16])
    _emit_subpass(_SPEC_LG_DEG, cid, sid, z1_ref, eslg, edlg, out_ref,
                  deg_ref, zblk, scr)
    _emit_subpass(_SPEC_G_DEG, cid, sid, z1_ref, esg, edg, out_ref,
                  deg_ref, zblk, scr)


def _k3_body(zblk, xf_ref, esglg, edglg, out_ref, *scr):
    cid = lax.axis_index("c")
    sid = lax.axis_index("s")
    _init_const_bufs(scr[15], scr[16])
    scr = list(scr[:21]) + [None] + list(scr[21:])
    _emit_subpass(_SPEC_GLG_B, cid, sid, xf_ref, esglg, edglg, out_ref,
                  None, zblk, scr)
    _emit_subpass(_SPEC_GLG_A, cid, sid, xf_ref, esglg, edglg, out_ref,
                  None, zblk, scr)


def _sc_scratch(with_deg):
    scr = [
        pltpu.VMEM((2 * _W,), _i32),
        pltpu.VMEM((2 * _W,), _i32),
        pltpu.VMEM((_RING,), _i32),
        pltpu.VMEM((_B,), _i32),
        pltpu.VMEM((_B,), _i32),
        pltpu.VMEM((_B,), _i32),
        pltpu.VMEM((_B,), _i32),
        pltpu.VMEM((_B,), _i32),
        pltpu.VMEM((_B,), _i32),
        pltpu.VMEM((_B,), _i32),
        pltpu.VMEM((_B,), _i32),
        pltpu.VMEM((_B, _D), _f32),
        pltpu.VMEM((_B, _D), _f32),
        pltpu.VMEM((_B, _D), _f32),
        pltpu.VMEM((_B, _D), _f32),
        pltpu.VMEM((_B,), _f32),
        pltpu.VMEM((1008,), _f32),
        pltpu.VMEM((40, _D), _f32),
        pltpu.VMEM((40, _D), _f32),
        pltpu.VMEM((1008,), _f32),
        pltpu.VMEM_SHARED((_ACC_ROWS, _D), _f32),
    ]
    if with_deg:
        scr.append(pltpu.VMEM_SHARED((_ACC_ROWS,), _f32))
    scr += [pltpu.SemaphoreType.DMA] * 8
    return scr


def _mesh():
    return plsc.VectorSubcoreMesh(core_axis_name="c", subcore_axis_name="s",
                                  num_cores=2, num_subcores=16)


_SC_PARAMS = pltpu.CompilerParams(needs_layout_passes=False)


def _glob_body(x_ref, o_ref):
    i = pl.program_id(0)

    @pl.when(i == 0)
    def _():
        o_ref[...] = jnp.zeros_like(o_ref)

    s = jnp.sum(x_ref[...], axis=0, keepdims=True)
    r = jnp.where(i < _N // _BLK, 0, 1)
    o_ref[pl.ds(r, 1), :] += s


def _glob_sums(xf):
    return pl.pallas_call(
        _glob_body,
        grid=(_R // _BLK,),
        in_specs=[pl.BlockSpec((_BLK, _D), lambda i: (i, 0))],
        out_specs=pl.BlockSpec((8, _D), lambda i: (0, 0)),
        out_shape=jax.ShapeDtypeStruct((8, _D), _f32),
    )(xf)


def _update_body(glob_ref, wcat_ref, w3_ref, ball_ref, xf_ref, y_ref, z1_ref,
                 deg_ref, out_ref):
    xf = xf_ref[...]
    cat = jnp.concatenate(
        [xf, y_ref[...], xf * deg_ref[...], z1_ref[...]], axis=1)
    acc = lax.dot_general(cat, wcat_ref[...], (((1,), (0,)), ((), ())),
                          preferred_element_type=_f32)
    cvec = lax.dot_general(glob_ref[...], w3_ref[...],
                           (((1,), (0,)), ((), ())),
                           preferred_element_type=_f32)
    out_ref[...] = acc + cvec + ball_ref[...]


def _update(xf, y, z1, deg, glob, wcat, w3, ball, row0, rows):
    blk0 = row0 // _BLK

    def rmap(i):
        return (i + blk0, 0)

    return pl.pallas_call(
        _update_body,
        grid=(rows // _BLK,),
        in_specs=[
            pl.BlockSpec((1, _D), lambda i: (0, 0)),
            pl.BlockSpec((4 * _D, _D), lambda i: (0, 0)),
            pl.BlockSpec((_D, _D), lambda i: (0, 0)),
            pl.BlockSpec((1, _D), lambda i: (0, 0)),
            pl.BlockSpec((_BLK, _D), rmap),
            pl.BlockSpec((_BLK, _D), rmap),
            pl.BlockSpec((_BLK, _D), rmap),
            pl.BlockSpec((_BLK, 1), rmap),
        ],
        out_specs=pl.BlockSpec((_BLK, _D), lambda i: (i, 0)),
        out_shape=jax.ShapeDtypeStruct((rows, _D), _f32),
    )(glob, wcat, w3, ball, xf, y, z1, deg)


def kernel(x_g, x_lg, edge_index_g, edge_index_lg, edge_index_glg,
           Wt_main, bt_main, Wt_list, bt_list,
           Wg_main, bg_main, Wg_list, bg_list):
    esg, edg = edge_index_g[0], edge_index_g[1]
    eslg, edlg = edge_index_lg[0], edge_index_lg[1]
    esglg, edglg = edge_index_glg[0], edge_index_glg[1]
    zblk = jnp.zeros((40, _D), _f32)

    z1 = pl.kernel(
        _k1_body,
        out_type=jax.ShapeDtypeStruct((_R, _D), _f32),
        mesh=_mesh(),
        scratch_types=_sc_scratch(False),
        compiler_params=_SC_PARAMS,
    )(zblk, x_g, x_lg, esg, edg, eslg, edlg)

    xf, deg = pl.kernel(
        _k2_body,
        out_type=(jax.ShapeDtypeStruct((_R, _D), _f32),
                  jax.ShapeDtypeStruct((_R,), _f32)),
        mesh=_mesh(),
        scratch_types=_sc_scratch(True),
        compiler_params=_SC_PARAMS,
    )(zblk, z1, esg, edg, eslg, edlg)

    y = pl.kernel(
        _k3_body,
        out_type=jax.ShapeDtypeStruct((_R, _D), _f32),
        mesh=_mesh(),
        scratch_types=_sc_scratch(False),
        compiler_params=_SC_PARAMS,
    )(zblk, xf, esglg, edglg)

    gs = _glob_sums(xf)
    glob_g = gs[0:1] / _N
    glob_lg = gs[1:2] / _M

    wcat_t = jnp.concatenate(
        [Wt_main[0] + Wt_list[1], Wt_main[1], Wt_main[2], Wt_list[0]], axis=0)
    ball_t = (bt_main.sum(0) + bt_list.sum(0))[None, :]
    wcat_g = jnp.concatenate(
        [Wg_main[0] + Wg_list[1], Wg_main[1], Wg_main[2], Wg_list[0]], axis=0)
    ball_g = (bg_main.sum(0) + bg_list.sum(0))[None, :]

    deg2 = deg[:, None]
    out_g = _update(xf, y, z1, deg2, glob_g, wcat_t, Wt_main[3], ball_t,
                    0, _N)
    out_lg = _update(xf, y, z1, deg2, glob_lg, wcat_g, Wg_main[3], ball_g,
                     _N, _M)
    return (out_g, out_lg)
